# baseline (speedup 1.0000x reference)
import functools

import jax
import jax.numpy as jnp
from jax import lax
from jax.experimental import pallas as pl
from jax.experimental.pallas import tpu as pltpu
from jax.experimental.pallas import tpu_sc as plsc

T = 8
N = 10000
E = 320000
ND = 128
KD = 16
H = 64

NC = 2
NS = 16
TPC = T // NC
EPT = E // NS
CHUNK = 1000
NCHUNK = EPT // CHUNK
SCH = 500
NCHT = EPT // SCH
NCHH = NCHT // 2
ROWS_PT = N // 10
ROWS16 = N // 16

_mesh = plsc.VectorSubcoreMesh(core_axis_name="c", subcore_axis_name="s",
                               num_cores=NC, num_subcores=NS)


@functools.partial(
    pl.kernel,
    out_type=jax.ShapeDtypeStruct((T * N,), jnp.float32),
    mesh=_mesh,
    compiler_params=pltpu.CompilerParams(use_tc_tiling_on_sc=False),
    scratch_types=[
        pltpu.VMEM((NCHH, SCH), jnp.int32),
        pltpu.VMEM((SCH,), jnp.float32),
        pltpu.VMEM_SHARED((N,), jnp.float32),
        pltpu.SemaphoreType.DMA,
    ],
)
def _sc_deg(ei_hbm, ones_hbm, zeros_hbm, deg_hbm, idx_v, ones_v, deg_sh, sem):
    cid = lax.axis_index("c")
    sid = lax.axis_index("s")
    pltpu.sync_copy(ones_hbm, ones_v)
    for tt in range(TPC):
        t = cid * TPC + tt
        @pl.when(sid < 10)
        def _():
            pltpu.sync_copy(zeros_hbm.at[pl.ds(sid * ROWS_PT, ROWS_PT)],
                            deg_sh.at[pl.ds(sid * ROWS_PT, ROWS_PT)])
        plsc.subcore_barrier()
        for h in range(2):
            pltpu.sync_copy(ei_hbm.at[1, t, sid, pl.ds(h * NCHH, NCHH)],
                            idx_v)
            for ci in range(NCHH):
                pltpu.sync_copy(ones_v, deg_sh.at[idx_v.at[ci]], add=True)
        plsc.subcore_barrier()
        @pl.when(sid < 10)
        def _():
            pltpu.sync_copy(deg_sh.at[pl.ds(sid * ROWS_PT, ROWS_PT)],
                            deg_hbm.at[pl.ds(t * N + sid * ROWS_PT, ROWS_PT)])
        plsc.subcore_barrier()


@functools.partial(
    pl.kernel,
    out_type=jax.ShapeDtypeStruct((T * N, H), jnp.float32),
    mesh=_mesh,
    compiler_params=pltpu.CompilerParams(use_tc_tiling_on_sc=False),
    scratch_types=[
        pltpu.VMEM((2, NCHH, SCH), jnp.int32),
        pltpu.VMEM((SCH, H), jnp.float32),
        pltpu.VMEM((SCH, H), jnp.float32),
        pltpu.VMEM_SHARED((N, H), jnp.float32),
        pltpu.SemaphoreType.DMA,
        pltpu.SemaphoreType.DMA,
        pltpu.SemaphoreType.DMA,
        pltpu.SemaphoreType.DMA,
    ],
)
def _sc_scatter(y_hbm, ei_hbm, acc_hbm, idx_v, rows_a, rows_b,
                acc_sh, gsem_a, gsem_b, ssem_a, ssem_b):
    cid = lax.axis_index("c")
    sid = lax.axis_index("s")
    rows_v = (rows_a, rows_b)
    gsem = (gsem_a, gsem_b)
    ssem = (ssem_a, ssem_b)

    def gather(ci, b):
        pltpu.async_copy(y_hbm.at[idx_v.at[0, ci]], rows_v[b], gsem[b])

    def gather_wait(ci, b):
        pltpu.make_async_copy(y_hbm.at[idx_v.at[0, ci]], rows_v[b],
                              gsem[b]).wait()

    def scat(ci, b):
        pltpu.async_copy(rows_v[b], acc_sh.at[idx_v.at[1, ci]], ssem[b],
                         add=True)

    def scat_wait(ci, b):
        pltpu.make_async_copy(rows_v[b], acc_sh.at[idx_v.at[1, ci]],
                              ssem[b]).wait()

    for tt in range(TPC):
        t = cid * TPC + tt
        pltpu.sync_copy(y_hbm.at[pl.ds(t * N + sid * ROWS16, ROWS16)],
                        acc_sh.at[pl.ds(sid * ROWS16, ROWS16)])
        plsc.subcore_barrier()
        for h in range(2):
            pltpu.sync_copy(ei_hbm.at[:, t, sid, pl.ds(h * NCHH, NCHH)],
                            idx_v)
            gather(0, 0)
            for ci in range(NCHH):
                b = ci & 1
                nb = 1 - b
                if ci + 1 < NCHH:
                    if ci >= 1:
                        scat_wait(ci - 1, nb)
                    gather(ci + 1, nb)
                gather_wait(ci, b)
                scat(ci, b)
            scat_wait(NCHH - 2, (NCHH - 2) & 1)
            scat_wait(NCHH - 1, (NCHH - 1) & 1)
        plsc.subcore_barrier()
        pltpu.sync_copy(acc_sh.at[pl.ds(sid * ROWS16, ROWS16)],
                        acc_hbm.at[pl.ds(t * N + sid * ROWS16, ROWS16)])
        plsc.subcore_barrier()


def _tc_xw_body(x_ref, w_ref, xw_ref):
    xw_ref[...] = jnp.dot(x_ref[...], w_ref[...],
                          preferred_element_type=jnp.float32)


def _tc_xw(x2, w):
    return pl.pallas_call(
        _tc_xw_body,
        grid=(T * N // CHUNK,),
        in_specs=[
            pl.BlockSpec((CHUNK, ND), lambda i: (i, 0)),
            pl.BlockSpec((ND, H), lambda i: (0, 0)),
        ],
        out_specs=pl.BlockSpec((CHUNK, H), lambda i: (i, 0)),
        out_shape=jax.ShapeDtypeStruct((T * N, H), jnp.float32),
    )(x2, w)


def _tc_scale_body(xw_ref, deg_ref, y_ref):
    dinv = lax.rsqrt(deg_ref[...] + 1.0)
    y_ref[...] = xw_ref[...] * jnp.reshape(dinv, (CHUNK, 1))


def _tc_scale(xw2, deg2):
    return pl.pallas_call(
        _tc_scale_body,
        grid=(T * N // CHUNK,),
        in_specs=[
            pl.BlockSpec((CHUNK, H), lambda i: (i, 0)),
            pl.BlockSpec((1, 1, CHUNK), lambda i: (i, 0, 0)),
        ],
        out_specs=pl.BlockSpec((CHUNK, H), lambda i: (i, 0)),
        out_shape=jax.ShapeDtypeStruct((T * N, H), jnp.float32),
    )(xw2, deg2)


def _tc_emb_body(acc_ref, deg_ref, b_ref, emb_ref):
    j = pl.program_id(1)
    dinv = lax.rsqrt(deg_ref[...] + 1.0)
    vals = jnp.tanh(acc_ref[...] * jnp.reshape(dinv, (CHUNK, 1)) + b_ref[...])
    colsum = jnp.sum(vals, axis=0, keepdims=True).reshape(1, 1, H)

    @pl.when(j == 0)
    def _():
        emb_ref[...] = jnp.zeros_like(emb_ref)

    emb_ref[...] += colsum * (1.0 / N)


def _tc_emb(acc2, deg2, b_gcn2):
    nj = N // CHUNK
    return pl.pallas_call(
        _tc_emb_body,
        grid=(T, nj),
        in_specs=[
            pl.BlockSpec((CHUNK, H), lambda t, j: (t * nj + j, 0)),
            pl.BlockSpec((1, 1, CHUNK), lambda t, j: (t * nj + j, 0, 0)),
            pl.BlockSpec((1, H), lambda t, j: (0, 0)),
        ],
        out_specs=pl.BlockSpec((1, 1, H), lambda t, j: (t, 0, 0)),
        out_shape=jax.ShapeDtypeStruct((T, 1, H), jnp.float32),
    )(acc2, deg2, b_gcn2)


def _tc_lstm_body(emb_ref, kpi_ref, wih_ref, whh_ref, bih_ref, bhh_ref,
                  whead_ref, bhead_ref, out_ref):
    emb = emb_ref[...]
    kpi = kpi_ref[...]
    wih = wih_ref[...]
    whh = whh_ref[...]
    bias = bih_ref[...] + bhh_ref[...]
    h = jnp.zeros((1, H), dtype=jnp.float32)
    c = jnp.zeros((1, H), dtype=jnp.float32)
    for t in range(T):
        xt = jnp.concatenate([emb[t:t + 1, :], kpi[t:t + 1, :]], axis=1)
        gates = (lax.dot_general(xt, wih, (((1,), (1,)), ((), ())),
                                 preferred_element_type=jnp.float32)
                 + lax.dot_general(h, whh, (((1,), (1,)), ((), ())),
                                   preferred_element_type=jnp.float32)
                 + bias)
        i = jax.nn.sigmoid(gates[:, 0 * H:1 * H])
        f = jax.nn.sigmoid(gates[:, 1 * H:2 * H])
        g = jnp.tanh(gates[:, 2 * H:3 * H])
        o = jax.nn.sigmoid(gates[:, 3 * H:4 * H])
        c = f * c + i * g
        h = o * jnp.tanh(c)
    s = jnp.sum(h * whead_ref[...], axis=1, keepdims=True)
    out_ref[...] = s + bhead_ref[...]


def _tc_lstm(emb, kpi2, w_ih, w_hh, b_ih2, b_hh2, w_head, b_head2):
    return pl.pallas_call(
        _tc_lstm_body,
        out_shape=jax.ShapeDtypeStruct((1, 1), jnp.float32),
    )(emb, kpi2, w_ih, w_hh, b_ih2, b_hh2, w_head, b_head2)


def kernel(x, edge_index, kpi_tensor, W_gcn, b_gcn, W_ih, W_hh, b_ih, b_hh,
           W_head, b_head):
    x2 = x.reshape(T * N, ND)
    src = edge_index[:, 0, :]
    dst = edge_index[:, 1, :]
    srcg = src + (jnp.arange(T, dtype=jnp.int32) * N)[:, None]
    ei5 = jnp.stack([srcg, dst]).reshape(2, T, NS, NCHT, SCH)
    ones_c = jnp.ones((SCH,), dtype=jnp.float32)
    zeros_n = jnp.zeros((N,), dtype=jnp.float32)

    xw2 = _tc_xw(x2, W_gcn)
    deg = _sc_deg(ei5, ones_c, zeros_n)
    deg2 = deg.reshape(T * N // CHUNK, 1, CHUNK)
    y2 = _tc_scale(xw2, deg2)
    acc2 = _sc_scatter(y2, ei5)
    emb = _tc_emb(acc2, deg2, b_gcn.reshape(1, H)).reshape(T, H)
    return _tc_lstm(emb, kpi_tensor.reshape(T, KD), W_ih, W_hh,
                    b_ih.reshape(1, 4 * H), b_hh.reshape(1, 4 * H),
                    W_head, b_head.reshape(1, 1))


# speedup vs baseline: 1.1408x; 1.1408x over previous
import functools

import jax
import jax.numpy as jnp
from jax import lax
from jax.experimental import pallas as pl
from jax.experimental.pallas import tpu as pltpu
from jax.experimental.pallas import tpu_sc as plsc

T = 8
N = 10000
E = 320000
ND = 128
KD = 16
H = 64

NC = 2
NS = 16
TPC = T // NC
EPT = E // NS
CHUNK = 1000
NCHUNK = EPT // CHUNK
SCH = 500
NCHT = EPT // SCH
NCHH = NCHT // 2
ROWS_PT = N // 10
ROWS16 = N // 16

_mesh = plsc.VectorSubcoreMesh(core_axis_name="c", subcore_axis_name="s",
                               num_cores=NC, num_subcores=NS)


@functools.partial(
    pl.kernel,
    out_type=jax.ShapeDtypeStruct((T * N,), jnp.float32),
    mesh=_mesh,
    compiler_params=pltpu.CompilerParams(use_tc_tiling_on_sc=False),
    scratch_types=[
        pltpu.VMEM((NCHH, SCH), jnp.int32),
        pltpu.VMEM((SCH,), jnp.float32),
        pltpu.VMEM_SHARED((N,), jnp.float32),
        pltpu.SemaphoreType.DMA,
    ],
)
def _sc_deg(ei_hbm, ones_hbm, zeros_hbm, deg_hbm, idx_v, ones_v, deg_sh, sem):
    cid = lax.axis_index("c")
    sid = lax.axis_index("s")
    pltpu.sync_copy(ones_hbm, ones_v)
    for tt in range(TPC):
        t = cid * TPC + tt
        @pl.when(sid < 10)
        def _():
            pltpu.sync_copy(zeros_hbm.at[pl.ds(sid * ROWS_PT, ROWS_PT)],
                            deg_sh.at[pl.ds(sid * ROWS_PT, ROWS_PT)])
        plsc.subcore_barrier()
        for h in range(2):
            pltpu.sync_copy(ei_hbm.at[t, 1, sid, pl.ds(h * NCHH, NCHH)],
                            idx_v)
            for ci in range(NCHH):
                pltpu.sync_copy(ones_v, deg_sh.at[idx_v.at[ci]], add=True)
        plsc.subcore_barrier()
        @pl.when(sid < 10)
        def _():
            pltpu.sync_copy(deg_sh.at[pl.ds(sid * ROWS_PT, ROWS_PT)],
                            deg_hbm.at[pl.ds(t * N + sid * ROWS_PT, ROWS_PT)])
        plsc.subcore_barrier()


@functools.partial(
    pl.kernel,
    out_type=jax.ShapeDtypeStruct((T, N, H), jnp.float32),
    mesh=_mesh,
    compiler_params=pltpu.CompilerParams(use_tc_tiling_on_sc=False),
    scratch_types=[
        pltpu.VMEM((2, NCHH, SCH), jnp.int32),
        pltpu.VMEM((SCH, H), jnp.float32),
        pltpu.VMEM((SCH, H), jnp.float32),
        pltpu.VMEM_SHARED((N, H), jnp.float32),
        pltpu.SemaphoreType.DMA,
        pltpu.SemaphoreType.DMA,
        pltpu.SemaphoreType.DMA,
        pltpu.SemaphoreType.DMA,
    ],
)
def _sc_scatter(y_hbm, ei_hbm, acc_hbm, idx_v, rows_a, rows_b,
                acc_sh, gsem_a, gsem_b, ssem_a, ssem_b):
    cid = lax.axis_index("c")
    sid = lax.axis_index("s")
    rows_v = (rows_a, rows_b)
    gsem = (gsem_a, gsem_b)
    ssem = (ssem_a, ssem_b)

    def gather(t, ci, b):
        pltpu.async_copy(y_hbm.at[t].at[idx_v.at[0, ci]], rows_v[b], gsem[b])

    def gather_wait(t, ci, b):
        pltpu.make_async_copy(y_hbm.at[t].at[idx_v.at[0, ci]], rows_v[b],
                              gsem[b]).wait()

    def scat(ci, b):
        pltpu.async_copy(rows_v[b], acc_sh.at[idx_v.at[1, ci]], ssem[b],
                         add=True)

    def scat_wait(ci, b):
        pltpu.make_async_copy(rows_v[b], acc_sh.at[idx_v.at[1, ci]],
                              ssem[b]).wait()

    for tt in range(TPC):
        t = cid * TPC + tt
        pltpu.sync_copy(y_hbm.at[t, pl.ds(sid * ROWS16, ROWS16)],
                        acc_sh.at[pl.ds(sid * ROWS16, ROWS16)])
        plsc.subcore_barrier()
        for h in range(2):
            pltpu.sync_copy(ei_hbm.at[t, :, sid, pl.ds(h * NCHH, NCHH)],
                            idx_v)
            gather(t, 0, 0)
            for ci in range(NCHH):
                b = ci & 1
                nb = 1 - b
                if ci + 1 < NCHH:
                    if ci >= 1:
                        scat_wait(ci - 1, nb)
                    gather(t, ci + 1, nb)
                gather_wait(t, ci, b)
                scat(ci, b)
            scat_wait(NCHH - 2, (NCHH - 2) & 1)
            scat_wait(NCHH - 1, (NCHH - 1) & 1)
        plsc.subcore_barrier()
        pltpu.sync_copy(acc_sh.at[pl.ds(sid * ROWS16, ROWS16)],
                        acc_hbm.at[t, pl.ds(sid * ROWS16, ROWS16)])
        plsc.subcore_barrier()


TBLK = 2000


def _tc_xw_body(x_ref, w_ref, xw_ref):
    xw_ref[...] = jnp.dot(x_ref[...], w_ref[...],
                          preferred_element_type=jnp.float32)


def _tc_xw(x2, w):
    return pl.pallas_call(
        _tc_xw_body,
        grid=(T * N // TBLK,),
        in_specs=[
            pl.BlockSpec((TBLK, ND), lambda i: (i, 0)),
            pl.BlockSpec((ND, H), lambda i: (0, 0)),
        ],
        out_specs=pl.BlockSpec((TBLK, H), lambda i: (i, 0)),
        out_shape=jax.ShapeDtypeStruct((T * N, H), jnp.float32),
    )(x2, w)


def _tc_scale_body(xw_ref, deg_ref, y_ref):
    dinv = lax.rsqrt(deg_ref[...] + 1.0)
    y_ref[...] = xw_ref[...] * jnp.reshape(dinv, (TBLK, 1))


def _tc_scale(xw2, deg2):
    return pl.pallas_call(
        _tc_scale_body,
        grid=(T * N // TBLK,),
        in_specs=[
            pl.BlockSpec((TBLK, H), lambda i: (i, 0)),
            pl.BlockSpec((1, 1, TBLK), lambda i: (i, 0, 0)),
        ],
        out_specs=pl.BlockSpec((TBLK, H), lambda i: (i, 0)),
        out_shape=jax.ShapeDtypeStruct((T * N, H), jnp.float32),
    )(xw2, deg2)


def _tc_emb_body(acc_ref, deg_ref, b_ref, emb_ref):
    j = pl.program_id(1)
    dinv = lax.rsqrt(deg_ref[...] + 1.0)
    vals = jnp.tanh(acc_ref[...] * jnp.reshape(dinv, (TBLK, 1)) + b_ref[...])
    colsum = jnp.sum(vals, axis=0, keepdims=True).reshape(1, 1, H)

    @pl.when(j == 0)
    def _():
        emb_ref[...] = jnp.zeros_like(emb_ref)

    emb_ref[...] += colsum * (1.0 / N)


def _tc_emb(acc2, deg2, b_gcn2):
    nj = N // TBLK
    return pl.pallas_call(
        _tc_emb_body,
        grid=(T, nj),
        in_specs=[
            pl.BlockSpec((TBLK, H), lambda t, j: (t * nj + j, 0)),
            pl.BlockSpec((1, 1, TBLK), lambda t, j: (t * nj + j, 0, 0)),
            pl.BlockSpec((1, H), lambda t, j: (0, 0)),
        ],
        out_specs=pl.BlockSpec((1, 1, H), lambda t, j: (t, 0, 0)),
        out_shape=jax.ShapeDtypeStruct((T, 1, H), jnp.float32),
    )(acc2, deg2, b_gcn2)


def _tc_lstm_body(emb_ref, kpi_ref, wih_ref, whh_ref, bih_ref, bhh_ref,
                  whead_ref, bhead_ref, out_ref):
    emb = emb_ref[...]
    kpi = kpi_ref[...]
    wih = wih_ref[...]
    whh = whh_ref[...]
    bias = bih_ref[...] + bhh_ref[...]
    h = jnp.zeros((1, H), dtype=jnp.float32)
    c = jnp.zeros((1, H), dtype=jnp.float32)
    for t in range(T):
        xt = jnp.concatenate([emb[t:t + 1, :], kpi[t:t + 1, :]], axis=1)
        gates = (lax.dot_general(xt, wih, (((1,), (1,)), ((), ())),
                                 preferred_element_type=jnp.float32)
                 + lax.dot_general(h, whh, (((1,), (1,)), ((), ())),
                                   preferred_element_type=jnp.float32)
                 + bias)
        i = jax.nn.sigmoid(gates[:, 0 * H:1 * H])
        f = jax.nn.sigmoid(gates[:, 1 * H:2 * H])
        g = jnp.tanh(gates[:, 2 * H:3 * H])
        o = jax.nn.sigmoid(gates[:, 3 * H:4 * H])
        c = f * c + i * g
        h = o * jnp.tanh(c)
    s = jnp.sum(h * whead_ref[...], axis=1, keepdims=True)
    out_ref[...] = s + bhead_ref[...]


def _tc_lstm(emb, kpi2, w_ih, w_hh, b_ih2, b_hh2, w_head, b_head2):
    return pl.pallas_call(
        _tc_lstm_body,
        out_shape=jax.ShapeDtypeStruct((1, 1), jnp.float32),
    )(emb, kpi2, w_ih, w_hh, b_ih2, b_hh2, w_head, b_head2)


def kernel(x, edge_index, kpi_tensor, W_gcn, b_gcn, W_ih, W_hh, b_ih, b_hh,
           W_head, b_head):
    x2 = x.reshape(T * N, ND)
    ei5 = edge_index.reshape(T, 2, NS, NCHT, SCH)
    ones_c = jnp.ones((SCH,), dtype=jnp.float32)
    zeros_n = jnp.zeros((N,), dtype=jnp.float32)

    xw2 = _tc_xw(x2, W_gcn)
    deg = _sc_deg(ei5, ones_c, zeros_n)
    deg2 = deg.reshape(T * N // TBLK, 1, TBLK)
    y2 = _tc_scale(xw2, deg2)
    acc2 = _sc_scatter(y2.reshape(T, N, H), ei5).reshape(T * N, H)
    emb = _tc_emb(acc2, deg2, b_gcn.reshape(1, H)).reshape(T, H)
    return _tc_lstm(emb, kpi_tensor.reshape(T, KD), W_ih, W_hh,
                    b_ih.reshape(1, 4 * H), b_hh.reshape(1, 4 * H),
                    W_head, b_head.reshape(1, 1))


# speedup vs baseline: 1.5966x; 1.3996x over previous
import functools

import jax
import jax.numpy as jnp
from jax import lax
from jax.experimental import pallas as pl
from jax.experimental.pallas import tpu as pltpu
from jax.experimental.pallas import tpu_sc as plsc

T = 8
N = 10000
E = 320000
ND = 128
KD = 16
H = 64

NC = 2
NS = 16
TPC = T // NC
EPT = E // NS
CHUNK = 1000
NCHUNK = EPT // CHUNK
SCH = 1000
NCHT = EPT // SCH
NCHH = NCHT // 2
ROWS_PT = N // 10
ROWS16 = N // 16

_mesh = plsc.VectorSubcoreMesh(core_axis_name="c", subcore_axis_name="s",
                               num_cores=NC, num_subcores=NS)


@functools.partial(
    pl.kernel,
    out_type=jax.ShapeDtypeStruct((T * N,), jnp.float32),
    mesh=_mesh,
    compiler_params=pltpu.CompilerParams(use_tc_tiling_on_sc=False),
    scratch_types=[
        pltpu.VMEM((NCHH, SCH), jnp.int32),
        pltpu.VMEM((SCH,), jnp.float32),
        pltpu.VMEM_SHARED((N,), jnp.float32),
        pltpu.SemaphoreType.DMA,
    ],
)
def _sc_deg(ei_hbm, ones_hbm, zeros_hbm, deg_hbm, idx_v, ones_v, deg_sh, sem):
    cid = lax.axis_index("c")
    sid = lax.axis_index("s")
    pltpu.sync_copy(ones_hbm, ones_v)
    for tt in range(TPC):
        t = cid * TPC + tt
        @pl.when(sid < 10)
        def _():
            pltpu.sync_copy(zeros_hbm.at[pl.ds(sid * ROWS_PT, ROWS_PT)],
                            deg_sh.at[pl.ds(sid * ROWS_PT, ROWS_PT)])
        plsc.subcore_barrier()
        for h in range(2):
            pltpu.sync_copy(ei_hbm.at[t, 1, sid, pl.ds(h * NCHH, NCHH)],
                            idx_v)
            for ci in range(NCHH):
                pltpu.sync_copy(ones_v, deg_sh.at[idx_v.at[ci]], add=True)
        plsc.subcore_barrier()
        @pl.when(sid < 10)
        def _():
            pltpu.sync_copy(deg_sh.at[pl.ds(sid * ROWS_PT, ROWS_PT)],
                            deg_hbm.at[pl.ds(t * N + sid * ROWS_PT, ROWS_PT)])
        plsc.subcore_barrier()


@functools.partial(
    pl.kernel,
    out_type=jax.ShapeDtypeStruct((T, N, H), jnp.bfloat16),
    mesh=_mesh,
    compiler_params=pltpu.CompilerParams(use_tc_tiling_on_sc=False),
    scratch_types=[
        pltpu.VMEM((2, NCHH, SCH), jnp.int32),
        pltpu.VMEM((SCH, H), jnp.bfloat16),
        pltpu.VMEM((SCH, H), jnp.bfloat16),
        pltpu.VMEM_SHARED((N, H), jnp.bfloat16),
        pltpu.SemaphoreType.DMA,
        pltpu.SemaphoreType.DMA,
        pltpu.SemaphoreType.DMA,
        pltpu.SemaphoreType.DMA,
    ],
)
def _sc_scatter(y_hbm, ei_hbm, acc_hbm, idx_v, rows_a, rows_b,
                acc_sh, gsem_a, gsem_b, ssem_a, ssem_b):
    cid = lax.axis_index("c")
    sid = lax.axis_index("s")
    rows_v = (rows_a, rows_b)
    gsem = (gsem_a, gsem_b)
    ssem = (ssem_a, ssem_b)

    def gather(t, ci, b):
        pltpu.async_copy(y_hbm.at[t].at[idx_v.at[0, ci]], rows_v[b], gsem[b])

    def gather_wait(t, ci, b):
        pltpu.make_async_copy(y_hbm.at[t].at[idx_v.at[0, ci]], rows_v[b],
                              gsem[b]).wait()

    def scat(ci, b):
        pltpu.async_copy(rows_v[b], acc_sh.at[idx_v.at[1, ci]], ssem[b],
                         add=True)

    def scat_wait(ci, b):
        pltpu.make_async_copy(rows_v[b], acc_sh.at[idx_v.at[1, ci]],
                              ssem[b]).wait()

    for tt in range(TPC):
        t = cid * TPC + tt
        pltpu.sync_copy(y_hbm.at[t, pl.ds(sid * ROWS16, ROWS16)],
                        acc_sh.at[pl.ds(sid * ROWS16, ROWS16)])
        plsc.subcore_barrier()
        for h in range(2):
            pltpu.sync_copy(ei_hbm.at[t, :, sid, pl.ds(h * NCHH, NCHH)],
                            idx_v)
            gather(t, 0, 0)
            for ci in range(NCHH):
                b = ci & 1
                nb = 1 - b
                if ci + 1 < NCHH:
                    if ci >= 1:
                        scat_wait(ci - 1, nb)
                    gather(t, ci + 1, nb)
                gather_wait(t, ci, b)
                scat(ci, b)
            scat_wait(NCHH - 2, (NCHH - 2) & 1)
            scat_wait(NCHH - 1, (NCHH - 1) & 1)
        plsc.subcore_barrier()
        pltpu.sync_copy(acc_sh.at[pl.ds(sid * ROWS16, ROWS16)],
                        acc_hbm.at[t, pl.ds(sid * ROWS16, ROWS16)])
        plsc.subcore_barrier()


TBLK = 2000


def _tc_xw_body(x_ref, w_ref, xw_ref):
    xw_ref[...] = jnp.dot(x_ref[...], w_ref[...],
                          preferred_element_type=jnp.float32)


def _tc_xw_body3(x_ref, w_ref, xw_ref):
    xw_ref[...] = jnp.dot(x_ref[0], w_ref[...],
                          preferred_element_type=jnp.float32)


def _tc_xw(x3, w):
    nj = N // TBLK
    return pl.pallas_call(
        _tc_xw_body3,
        grid=(T, nj),
        in_specs=[
            pl.BlockSpec((1, TBLK, ND), lambda t, j: (t, j, 0)),
            pl.BlockSpec((ND, H), lambda t, j: (0, 0)),
        ],
        out_specs=pl.BlockSpec((TBLK, H), lambda t, j: (t * nj + j, 0)),
        out_shape=jax.ShapeDtypeStruct((T * N, H), jnp.float32),
    )(x3, w)


def _tc_scale_body(xw_ref, deg_ref, y_ref):
    dinv = lax.rsqrt(deg_ref[...] + 1.0)
    y_ref[...] = (xw_ref[...] * jnp.reshape(dinv, (TBLK, 1))
                  ).astype(jnp.bfloat16)


def _tc_scale(xw2, deg2):
    return pl.pallas_call(
        _tc_scale_body,
        grid=(T * N // TBLK,),
        in_specs=[
            pl.BlockSpec((TBLK, H), lambda i: (i, 0)),
            pl.BlockSpec((1, 1, TBLK), lambda i: (i, 0, 0)),
        ],
        out_specs=pl.BlockSpec((TBLK, H), lambda i: (i, 0)),
        out_shape=jax.ShapeDtypeStruct((T * N, H), jnp.bfloat16),
    )(xw2, deg2)


def _tc_emb_body(acc_ref, deg_ref, b_ref, emb_ref):
    j = pl.program_id(1)
    dinv = lax.rsqrt(deg_ref[...] + 1.0)
    vals = jnp.tanh(acc_ref[...].astype(jnp.float32)
                    * jnp.reshape(dinv, (TBLK, 1)) + b_ref[...])
    colsum = jnp.sum(vals, axis=0, keepdims=True).reshape(1, 1, H)

    @pl.when(j == 0)
    def _():
        emb_ref[...] = jnp.zeros_like(emb_ref)

    emb_ref[...] += colsum * (1.0 / N)


def _tc_emb(acc2, deg2, b_gcn2):
    nj = N // TBLK
    return pl.pallas_call(
        _tc_emb_body,
        grid=(T, nj),
        in_specs=[
            pl.BlockSpec((TBLK, H), lambda t, j: (t * nj + j, 0)),
            pl.BlockSpec((1, 1, TBLK), lambda t, j: (t * nj + j, 0, 0)),
            pl.BlockSpec((1, H), lambda t, j: (0, 0)),
        ],
        out_specs=pl.BlockSpec((1, 1, H), lambda t, j: (t, 0, 0)),
        out_shape=jax.ShapeDtypeStruct((T, 1, H), jnp.float32),
    )(acc2, deg2, b_gcn2)


def _tc_lstm_body(emb_ref, kpi_ref, wih_ref, whh_ref, bih_ref, bhh_ref,
                  whead_ref, bhead_ref, out_ref):
    emb = emb_ref[...]
    kpi = kpi_ref[...]
    wih = wih_ref[...]
    whh = whh_ref[...]
    bias = bih_ref[...] + bhh_ref[...]
    h = jnp.zeros((1, H), dtype=jnp.float32)
    c = jnp.zeros((1, H), dtype=jnp.float32)
    for t in range(T):
        xt = jnp.concatenate([emb[t:t + 1, :], kpi[t:t + 1, :]], axis=1)
        gates = (lax.dot_general(xt, wih, (((1,), (1,)), ((), ())),
                                 preferred_element_type=jnp.float32)
                 + lax.dot_general(h, whh, (((1,), (1,)), ((), ())),
                                   preferred_element_type=jnp.float32)
                 + bias)
        i = jax.nn.sigmoid(gates[:, 0 * H:1 * H])
        f = jax.nn.sigmoid(gates[:, 1 * H:2 * H])
        g = jnp.tanh(gates[:, 2 * H:3 * H])
        o = jax.nn.sigmoid(gates[:, 3 * H:4 * H])
        c = f * c + i * g
        h = o * jnp.tanh(c)
    s = jnp.sum(h * whead_ref[...], axis=1, keepdims=True)
    out_ref[...] = s + bhead_ref[...]


def _tc_lstm(emb, kpi2, w_ih, w_hh, b_ih2, b_hh2, w_head, b_head2):
    return pl.pallas_call(
        _tc_lstm_body,
        out_shape=jax.ShapeDtypeStruct((1, 1), jnp.float32),
    )(emb, kpi2, w_ih, w_hh, b_ih2, b_hh2, w_head, b_head2)


def kernel(x, edge_index, kpi_tensor, W_gcn, b_gcn, W_ih, W_hh, b_ih, b_hh,
           W_head, b_head):
    ei5 = edge_index.reshape(T, 2, NS, NCHT, SCH)
    ones_c = jnp.ones((SCH,), dtype=jnp.float32)
    zeros_n = jnp.zeros((N,), dtype=jnp.float32)

    xw2 = _tc_xw(x, W_gcn)
    deg = _sc_deg(ei5, ones_c, zeros_n)
    deg2 = deg.reshape(T * N // TBLK, 1, TBLK)
    y2 = _tc_scale(xw2, deg2)
    acc2 = _sc_scatter(y2.reshape(T, N, H), ei5).reshape(T * N, H)
    emb = _tc_emb(acc2, deg2, b_gcn.reshape(1, H)).reshape(T, H)
    return _tc_lstm(emb, kpi_tensor.reshape(T, KD), W_ih, W_hh,
                    b_ih.reshape(1, 4 * H), b_hh.reshape(1, 4 * H),
                    W_head, b_head.reshape(1, 1))


# speedup vs baseline: 1.8210x; 1.1405x over previous
import functools

import jax
import jax.numpy as jnp
from jax import lax
from jax.experimental import pallas as pl
from jax.experimental.pallas import tpu as pltpu
from jax.experimental.pallas import tpu_sc as plsc

T = 8
N = 10000
E = 320000
ND = 128
KD = 16
H = 64

NC = 2
NS = 16
TPC = T // NC
EPT = E // NS
CHUNK = 1000
NCHUNK = EPT // CHUNK
SCH = 1000
NCHT = EPT // SCH
NCHH = NCHT // 2
ROWS_PT = N // 10
ROWS16 = N // 16

_mesh = plsc.VectorSubcoreMesh(core_axis_name="c", subcore_axis_name="s",
                               num_cores=NC, num_subcores=NS)


@functools.partial(
    pl.kernel,
    out_type=jax.ShapeDtypeStruct((T * N,), jnp.float32),
    mesh=_mesh,
    compiler_params=pltpu.CompilerParams(use_tc_tiling_on_sc=False),
    scratch_types=[
        pltpu.VMEM((NCHH, SCH), jnp.int32),
        pltpu.VMEM((SCH,), jnp.float32),
        pltpu.VMEM_SHARED((N,), jnp.float32),
        pltpu.SemaphoreType.DMA,
    ],
)
def _sc_deg(ei_hbm, ones_hbm, zeros_hbm, deg_hbm, idx_v, ones_v, deg_sh, sem):
    cid = lax.axis_index("c")
    sid = lax.axis_index("s")
    pltpu.sync_copy(ones_hbm, ones_v)
    for tt in range(TPC):
        t = cid * TPC + tt
        @pl.when(sid < 10)
        def _():
            pltpu.sync_copy(zeros_hbm.at[pl.ds(sid * ROWS_PT, ROWS_PT)],
                            deg_sh.at[pl.ds(sid * ROWS_PT, ROWS_PT)])
        plsc.subcore_barrier()
        for h in range(2):
            pltpu.sync_copy(ei_hbm.at[t, 1, sid, pl.ds(h * NCHH, NCHH)],
                            idx_v)
            for ci in range(NCHH):
                pltpu.sync_copy(ones_v, deg_sh.at[idx_v.at[ci]], add=True)
        plsc.subcore_barrier()
        @pl.when(sid < 10)
        def _():
            pltpu.sync_copy(deg_sh.at[pl.ds(sid * ROWS_PT, ROWS_PT)],
                            deg_hbm.at[pl.ds(t * N + sid * ROWS_PT, ROWS_PT)])
        plsc.subcore_barrier()


def _make_scatter(toff):
    @functools.partial(
        pl.kernel,
        out_type=jax.ShapeDtypeStruct((2 * NC, N, H), jnp.bfloat16),
        mesh=_mesh,
        compiler_params=pltpu.CompilerParams(use_tc_tiling_on_sc=False),
        scratch_types=[
            pltpu.VMEM((2, NCHH, SCH), jnp.int32),
            pltpu.VMEM((SCH, H), jnp.bfloat16),
            pltpu.VMEM((SCH, H), jnp.bfloat16),
            pltpu.VMEM_SHARED((N, H), jnp.bfloat16),
            pltpu.SemaphoreType.DMA,
            pltpu.SemaphoreType.DMA,
            pltpu.SemaphoreType.DMA,
            pltpu.SemaphoreType.DMA,
        ],
    )
    def _sc_scatter(y_hbm, ei_hbm, acc_hbm, idx_v, rows_a, rows_b,
                    acc_sh, gsem_a, gsem_b, ssem_a, ssem_b):
        cid = lax.axis_index("c")
        sid = lax.axis_index("s")
        rows_v = (rows_a, rows_b)
        gsem = (gsem_a, gsem_b)
        ssem = (ssem_a, ssem_b)

        def gather(tl, ci, b):
            pltpu.async_copy(y_hbm.at[tl].at[idx_v.at[0, ci]], rows_v[b],
                             gsem[b])

        def gather_wait(tl, ci, b):
            pltpu.make_async_copy(y_hbm.at[tl].at[idx_v.at[0, ci]], rows_v[b],
                                  gsem[b]).wait()

        def scat(ci, b):
            pltpu.async_copy(rows_v[b], acc_sh.at[idx_v.at[1, ci]], ssem[b],
                             add=True)

        def scat_wait(ci, b):
            pltpu.make_async_copy(rows_v[b], acc_sh.at[idx_v.at[1, ci]],
                                  ssem[b]).wait()

        for tt in range(2):
            tl = cid * 2 + tt
            tg = cid * TPC + toff + tt
            pltpu.sync_copy(y_hbm.at[tl, pl.ds(sid * ROWS16, ROWS16)],
                            acc_sh.at[pl.ds(sid * ROWS16, ROWS16)])
            plsc.subcore_barrier()
            for h in range(2):
                pltpu.sync_copy(ei_hbm.at[tg, :, sid, pl.ds(h * NCHH, NCHH)],
                                idx_v)
                gather(tl, 0, 0)
                for ci in range(NCHH):
                    b = ci & 1
                    nb = 1 - b
                    if ci + 1 < NCHH:
                        if ci >= 1:
                            scat_wait(ci - 1, nb)
                        gather(tl, ci + 1, nb)
                    gather_wait(tl, ci, b)
                    scat(ci, b)
                scat_wait(NCHH - 2, (NCHH - 2) & 1)
                scat_wait(NCHH - 1, (NCHH - 1) & 1)
            plsc.subcore_barrier()
            pltpu.sync_copy(acc_sh.at[pl.ds(sid * ROWS16, ROWS16)],
                            acc_hbm.at[tl, pl.ds(sid * ROWS16, ROWS16)])
            plsc.subcore_barrier()
    return _sc_scatter


_sc_scatter_a = _make_scatter(0)
_sc_scatter_b = _make_scatter(2)


TBLK = 2000


def _tc_xw_body(x_ref, w_ref, xw_ref):
    xw_ref[...] = jnp.dot(x_ref[...], w_ref[...],
                          preferred_element_type=jnp.float32)


def _tc_xw_body3(x_ref, w_ref, xw_ref):
    xw_ref[...] = jnp.dot(x_ref[0], w_ref[...],
                          preferred_element_type=jnp.float32)


def _tc_xw(x3, w):
    nj = N // TBLK
    return pl.pallas_call(
        _tc_xw_body3,
        grid=(T, nj),
        in_specs=[
            pl.BlockSpec((1, TBLK, ND), lambda t, j: (t, j, 0)),
            pl.BlockSpec((ND, H), lambda t, j: (0, 0)),
        ],
        out_specs=pl.BlockSpec((TBLK, H), lambda t, j: (t * nj + j, 0)),
        out_shape=jax.ShapeDtypeStruct((T * N, H), jnp.float32),
    )(x3, w)


def _tc_scale_body(xw_ref, deg_ref, y_ref):
    dinv = lax.rsqrt(deg_ref[...] + 1.0)
    y_ref[...] = (xw_ref[...] * jnp.reshape(dinv, (TBLK, 1))
                  ).astype(jnp.bfloat16)


def _tc_scale(xw2, deg2, toff):
    nj = N // TBLK

    def tmap(ti):
        return ti % 2 + (ti // 2) * 4 + toff

    return pl.pallas_call(
        _tc_scale_body,
        grid=(2 * NC, nj),
        in_specs=[
            pl.BlockSpec((TBLK, H), lambda ti, j: (tmap(ti) * nj + j, 0)),
            pl.BlockSpec((1, 1, TBLK), lambda ti, j: (tmap(ti) * nj + j, 0, 0)),
        ],
        out_specs=pl.BlockSpec((TBLK, H), lambda ti, j: (ti * nj + j, 0)),
        out_shape=jax.ShapeDtypeStruct((2 * NC * N, H), jnp.bfloat16),
    )(xw2, deg2)


def _tc_emb_body(acc_ref, deg_ref, b_ref, emb_ref):
    j = pl.program_id(1)
    dinv = lax.rsqrt(deg_ref[...] + 1.0)
    acc = acc_ref[...].astype(jnp.float32)
    vals = jnp.tanh(acc * jnp.reshape(dinv, (TBLK, 1)) + b_ref[...])
    colsum = jnp.sum(vals, axis=0, keepdims=True).reshape(1, 1, H)

    @pl.when(j == 0)
    def _():
        emb_ref[...] = jnp.zeros_like(emb_ref)

    emb_ref[...] += colsum * (1.0 / N)


def _tc_emb(acc2, deg2, b_gcn2, toff):
    nj = N // TBLK

    def tmap(ti):
        return ti % 2 + (ti // 2) * 4 + toff

    return pl.pallas_call(
        _tc_emb_body,
        grid=(2 * NC, nj),
        in_specs=[
            pl.BlockSpec((TBLK, H), lambda ti, j: (ti * nj + j, 0)),
            pl.BlockSpec((1, 1, TBLK), lambda ti, j: (tmap(ti) * nj + j, 0, 0)),
            pl.BlockSpec((1, H), lambda ti, j: (0, 0)),
        ],
        out_specs=pl.BlockSpec((1, 1, H), lambda ti, j: (ti, 0, 0)),
        out_shape=jax.ShapeDtypeStruct((2 * NC, 1, H), jnp.float32),
    )(acc2, deg2, b_gcn2)


def _tc_lstm_body(emba_ref, embb_ref, kpi_ref, wih_ref, whh_ref, bih_ref,
                  bhh_ref, whead_ref, bhead_ref, out_ref):
    emba = emba_ref[...]
    embb = embb_ref[...]
    rows = {0: emba[0:1], 1: emba[1:2], 2: embb[0:1], 3: embb[1:2],
            4: emba[2:3], 5: emba[3:4], 6: embb[2:3], 7: embb[3:4]}
    emb = jnp.concatenate([rows[t] for t in range(T)], axis=0)
    kpi = kpi_ref[...]
    wih = wih_ref[...]
    whh = whh_ref[...]
    bias = bih_ref[...] + bhh_ref[...]
    h = jnp.zeros((1, H), dtype=jnp.float32)
    c = jnp.zeros((1, H), dtype=jnp.float32)
    for t in range(T):
        xt = jnp.concatenate([emb[t:t + 1, :], kpi[t:t + 1, :]], axis=1)
        gates = (lax.dot_general(xt, wih, (((1,), (1,)), ((), ())),
                                 preferred_element_type=jnp.float32)
                 + lax.dot_general(h, whh, (((1,), (1,)), ((), ())),
                                   preferred_element_type=jnp.float32)
                 + bias)
        i = jax.nn.sigmoid(gates[:, 0 * H:1 * H])
        f = jax.nn.sigmoid(gates[:, 1 * H:2 * H])
        g = jnp.tanh(gates[:, 2 * H:3 * H])
        o = jax.nn.sigmoid(gates[:, 3 * H:4 * H])
        c = f * c + i * g
        h = o * jnp.tanh(c)
    s = jnp.sum(h * whead_ref[...], axis=1, keepdims=True)
    out_ref[...] = s + bhead_ref[...]


def _tc_lstm(emba, embb, kpi2, w_ih, w_hh, b_ih2, b_hh2, w_head, b_head2):
    return pl.pallas_call(
        _tc_lstm_body,
        out_shape=jax.ShapeDtypeStruct((1, 1), jnp.float32),
    )(emba, embb, kpi2, w_ih, w_hh, b_ih2, b_hh2, w_head, b_head2)


def kernel(x, edge_index, kpi_tensor, W_gcn, b_gcn, W_ih, W_hh, b_ih, b_hh,
           W_head, b_head):
    ei5 = edge_index.reshape(T, 2, NS, NCHT, SCH)
    ones_c = jnp.ones((SCH,), dtype=jnp.float32)
    zeros_n = jnp.zeros((N,), dtype=jnp.float32)

    xw2 = _tc_xw(x, W_gcn)
    deg = _sc_deg(ei5, ones_c, zeros_n)
    deg2 = deg.reshape(T * N // TBLK, 1, TBLK)
    b2 = b_gcn.reshape(1, H)
    ya = _tc_scale(xw2, deg2, 0)
    acca = _sc_scatter_a(ya.reshape(2 * NC, N, H), ei5)
    yb = _tc_scale(xw2, deg2, 2)
    accb = _sc_scatter_b(yb.reshape(2 * NC, N, H), ei5)
    emba = _tc_emb(acca.reshape(2 * NC * N, H), deg2, b2, 0)
    embb = _tc_emb(accb.reshape(2 * NC * N, H), deg2, b2, 2)
    return _tc_lstm(emba.reshape(2 * NC, H), embb.reshape(2 * NC, H),
                    kpi_tensor.reshape(T, KD), W_ih, W_hh,
                    b_ih.reshape(1, 4 * H), b_hh.reshape(1, 4 * H),
                    W_head, b_head.reshape(1, 1))


# speedup vs baseline: 2.0188x; 1.1086x over previous
import functools

import jax
import jax.numpy as jnp
from jax import lax
from jax.experimental import pallas as pl
from jax.experimental.pallas import tpu as pltpu
from jax.experimental.pallas import tpu_sc as plsc

T = 8
N = 10000
E = 320000
ND = 128
KD = 16
H = 64

NC = 2
NS = 16
TPC = T // NC
EPT = E // NS
CHUNK = 1000
NCHUNK = EPT // CHUNK
SCH = 1000
NCHT = EPT // SCH
NCHH = NCHT // 2
ROWS_PT = N // 10
ROWS16 = N // 16

_mesh = plsc.VectorSubcoreMesh(core_axis_name="c", subcore_axis_name="s",
                               num_cores=NC, num_subcores=NS)


@functools.partial(
    pl.kernel,
    out_type=jax.ShapeDtypeStruct((T * N,), jnp.float32),
    mesh=_mesh,
    compiler_params=pltpu.CompilerParams(use_tc_tiling_on_sc=False),
    scratch_types=[
        pltpu.VMEM((NCHH, SCH), jnp.int32),
        pltpu.VMEM((SCH,), jnp.float32),
        pltpu.VMEM_SHARED((N,), jnp.float32),
        pltpu.SemaphoreType.DMA,
    ],
)
def _sc_deg(ei_hbm, ones_hbm, zeros_hbm, deg_hbm, idx_v, ones_v, deg_sh, sem):
    cid = lax.axis_index("c")
    sid = lax.axis_index("s")
    pltpu.sync_copy(ones_hbm, ones_v)
    for tt in range(TPC):
        t = cid * TPC + tt
        @pl.when(sid < 10)
        def _():
            pltpu.sync_copy(zeros_hbm.at[pl.ds(sid * ROWS_PT, ROWS_PT)],
                            deg_sh.at[pl.ds(sid * ROWS_PT, ROWS_PT)])
        plsc.subcore_barrier()
        for h in range(2):
            pltpu.sync_copy(ei_hbm.at[t, 1, sid, pl.ds(h * NCHH, NCHH)],
                            idx_v)
            for ci in range(NCHH):
                pltpu.sync_copy(ones_v, deg_sh.at[idx_v.at[ci]], add=True)
        plsc.subcore_barrier()
        @pl.when(sid < 10)
        def _():
            pltpu.sync_copy(deg_sh.at[pl.ds(sid * ROWS_PT, ROWS_PT)],
                            deg_hbm.at[pl.ds(t * N + sid * ROWS_PT, ROWS_PT)])
        plsc.subcore_barrier()


def _make_scatter(toff):
    @functools.partial(
        pl.kernel,
        out_type=jax.ShapeDtypeStruct((NC, N, H), jnp.bfloat16),
        mesh=_mesh,
        compiler_params=pltpu.CompilerParams(use_tc_tiling_on_sc=False),
        scratch_types=[
            pltpu.VMEM((2, NCHH, SCH), jnp.int32),
            pltpu.VMEM((SCH, H), jnp.bfloat16),
            pltpu.VMEM((SCH, H), jnp.bfloat16),
            pltpu.VMEM_SHARED((N, H), jnp.bfloat16),
            pltpu.SemaphoreType.DMA,
            pltpu.SemaphoreType.DMA,
            pltpu.SemaphoreType.DMA,
            pltpu.SemaphoreType.DMA,
        ],
    )
    def _sc_scatter(y_hbm, ei_hbm, acc_hbm, idx_v, rows_a, rows_b,
                    acc_sh, gsem_a, gsem_b, ssem_a, ssem_b):
        cid = lax.axis_index("c")
        sid = lax.axis_index("s")
        rows_v = (rows_a, rows_b)
        gsem = (gsem_a, gsem_b)
        ssem = (ssem_a, ssem_b)

        def gather(tl, ci, b):
            pltpu.async_copy(y_hbm.at[tl].at[idx_v.at[0, ci]], rows_v[b],
                             gsem[b])

        def gather_wait(tl, ci, b):
            pltpu.make_async_copy(y_hbm.at[tl].at[idx_v.at[0, ci]], rows_v[b],
                                  gsem[b]).wait()

        def scat(ci, b):
            pltpu.async_copy(rows_v[b], acc_sh.at[idx_v.at[1, ci]], ssem[b],
                             add=True)

        def scat_wait(ci, b):
            pltpu.make_async_copy(rows_v[b], acc_sh.at[idx_v.at[1, ci]],
                                  ssem[b]).wait()

        for tt in range(1):
            tl = cid
            tg = cid * TPC + toff + tt
            pltpu.sync_copy(y_hbm.at[tl, pl.ds(sid * ROWS16, ROWS16)],
                            acc_sh.at[pl.ds(sid * ROWS16, ROWS16)])
            plsc.subcore_barrier()
            for h in range(2):
                pltpu.sync_copy(ei_hbm.at[tg, :, sid, pl.ds(h * NCHH, NCHH)],
                                idx_v)
                gather(tl, 0, 0)
                for ci in range(NCHH):
                    b = ci & 1
                    nb = 1 - b
                    if ci + 1 < NCHH:
                        if ci >= 1:
                            scat_wait(ci - 1, nb)
                        gather(tl, ci + 1, nb)
                    gather_wait(tl, ci, b)
                    scat(ci, b)
                scat_wait(NCHH - 2, (NCHH - 2) & 1)
                scat_wait(NCHH - 1, (NCHH - 1) & 1)
            plsc.subcore_barrier()
            pltpu.sync_copy(acc_sh.at[pl.ds(sid * ROWS16, ROWS16)],
                            acc_hbm.at[tl, pl.ds(sid * ROWS16, ROWS16)])
            plsc.subcore_barrier()
    return _sc_scatter


_sc_scatters = [_make_scatter(g) for g in range(TPC)]


TBLK = 2000


def _tc_xw_body(x_ref, w_ref, xw_ref):
    xw_ref[...] = jnp.dot(x_ref[...], w_ref[...],
                          preferred_element_type=jnp.float32)


def _tc_xw_body3(x_ref, w_ref, xw_ref):
    xw_ref[...] = jnp.dot(x_ref[0], w_ref[...],
                          preferred_element_type=jnp.float32)


def _tc_xw(x3, w):
    nj = N // TBLK
    return pl.pallas_call(
        _tc_xw_body3,
        grid=(T, nj),
        in_specs=[
            pl.BlockSpec((1, TBLK, ND), lambda t, j: (t, j, 0)),
            pl.BlockSpec((ND, H), lambda t, j: (0, 0)),
        ],
        out_specs=pl.BlockSpec((TBLK, H), lambda t, j: (t * nj + j, 0)),
        out_shape=jax.ShapeDtypeStruct((T * N, H), jnp.float32),
    )(x3, w)


def _tc_scale_body(xw_ref, deg_ref, y_ref):
    dinv = lax.rsqrt(deg_ref[...] + 1.0)
    y_ref[...] = (xw_ref[...] * jnp.reshape(dinv, (TBLK, 1))
                  ).astype(jnp.bfloat16)


def _tc_scale(xw2, deg2, toff):
    nj = N // TBLK

    def tmap(ti):
        return ti * TPC + toff

    return pl.pallas_call(
        _tc_scale_body,
        grid=(NC, nj),
        in_specs=[
            pl.BlockSpec((TBLK, H), lambda ti, j: (tmap(ti) * nj + j, 0)),
            pl.BlockSpec((1, 1, TBLK), lambda ti, j: (tmap(ti) * nj + j, 0, 0)),
        ],
        out_specs=pl.BlockSpec((TBLK, H), lambda ti, j: (ti * nj + j, 0)),
        out_shape=jax.ShapeDtypeStruct((NC * N, H), jnp.bfloat16),
    )(xw2, deg2)


def _tc_emb_body(acc_ref, deg_ref, b_ref, emb_ref):
    j = pl.program_id(1)
    dinv = lax.rsqrt(deg_ref[...] + 1.0)
    acc = acc_ref[...].astype(jnp.float32)
    vals = jnp.tanh(acc * jnp.reshape(dinv, (TBLK, 1)) + b_ref[...])
    colsum = jnp.sum(vals, axis=0, keepdims=True).reshape(1, 1, H)

    @pl.when(j == 0)
    def _():
        emb_ref[...] = jnp.zeros_like(emb_ref)

    emb_ref[...] += colsum * (1.0 / N)


def _tc_emb(acc2, deg2, b_gcn2, toff):
    nj = N // TBLK

    def tmap(ti):
        return ti * TPC + toff

    return pl.pallas_call(
        _tc_emb_body,
        grid=(NC, nj),
        in_specs=[
            pl.BlockSpec((TBLK, H), lambda ti, j: (ti * nj + j, 0)),
            pl.BlockSpec((1, 1, TBLK), lambda ti, j: (tmap(ti) * nj + j, 0, 0)),
            pl.BlockSpec((1, H), lambda ti, j: (0, 0)),
        ],
        out_specs=pl.BlockSpec((1, 1, H), lambda ti, j: (ti, 0, 0)),
        out_shape=jax.ShapeDtypeStruct((NC, 1, H), jnp.float32),
    )(acc2, deg2, b_gcn2)


def _tc_lstm_body(emb0_ref, emb1_ref, emb2_ref, emb3_ref, kpi_ref, wih_ref,
                  whh_ref, bih_ref, bhh_ref, whead_ref, bhead_ref, out_ref):
    groups = [emb0_ref[...], emb1_ref[...], emb2_ref[...], emb3_ref[...]]
    emb = jnp.concatenate(
        [groups[t % TPC][t // TPC:t // TPC + 1] for t in range(T)], axis=0)
    kpi = kpi_ref[...]
    wih = wih_ref[...]
    whh = whh_ref[...]
    bias = bih_ref[...] + bhh_ref[...]
    h = jnp.zeros((1, H), dtype=jnp.float32)
    c = jnp.zeros((1, H), dtype=jnp.float32)
    for t in range(T):
        xt = jnp.concatenate([emb[t:t + 1, :], kpi[t:t + 1, :]], axis=1)
        gates = (lax.dot_general(xt, wih, (((1,), (1,)), ((), ())),
                                 preferred_element_type=jnp.float32)
                 + lax.dot_general(h, whh, (((1,), (1,)), ((), ())),
                                   preferred_element_type=jnp.float32)
                 + bias)
        i = jax.nn.sigmoid(gates[:, 0 * H:1 * H])
        f = jax.nn.sigmoid(gates[:, 1 * H:2 * H])
        g = jnp.tanh(gates[:, 2 * H:3 * H])
        o = jax.nn.sigmoid(gates[:, 3 * H:4 * H])
        c = f * c + i * g
        h = o * jnp.tanh(c)
    s = jnp.sum(h * whead_ref[...], axis=1, keepdims=True)
    out_ref[...] = s + bhead_ref[...]


def _tc_lstm(embs, kpi2, w_ih, w_hh, b_ih2, b_hh2, w_head, b_head2):
    return pl.pallas_call(
        _tc_lstm_body,
        out_shape=jax.ShapeDtypeStruct((1, 1), jnp.float32),
    )(*embs, kpi2, w_ih, w_hh, b_ih2, b_hh2, w_head, b_head2)


def kernel(x, edge_index, kpi_tensor, W_gcn, b_gcn, W_ih, W_hh, b_ih, b_hh,
           W_head, b_head):
    ei5 = edge_index.reshape(T, 2, NS, NCHT, SCH)
    ones_c = jnp.ones((SCH,), dtype=jnp.float32)
    zeros_n = jnp.zeros((N,), dtype=jnp.float32)

    xw2 = _tc_xw(x, W_gcn)
    deg = _sc_deg(ei5, ones_c, zeros_n)
    deg2 = deg.reshape(T * N // TBLK, 1, TBLK)
    b2 = b_gcn.reshape(1, H)
    accs = []
    for g in range(TPC):
        yg = _tc_scale(xw2, deg2, g)
        accs.append(_sc_scatters[g](yg.reshape(NC, N, H), ei5))
    embs = [_tc_emb(accs[g].reshape(NC * N, H), deg2, b2, g).reshape(NC, H)
            for g in range(TPC)]
    return _tc_lstm(embs, kpi_tensor.reshape(T, KD), W_ih, W_hh,
                    b_ih.reshape(1, 4 * H), b_hh.reshape(1, 4 * H),
                    W_head, b_head.reshape(1, 1))
